# baseline (speedup 1.0000x reference)
import jax
import jax.numpy as jnp
import numpy as np
from jax import lax
from jax.experimental import pallas as pl
from jax.experimental.pallas import tpu as pltpu
from jax.experimental.pallas import tpu_sc as plsc

B, S, D = 16, 4096, 128
OBS, C, A = 39, 32, 4
NC, NS, L = 2, 16, 16
NW = NC * NS
SC_ROWS = 4
TC_ROWS = B - SC_ROWS
WPR = NW // SC_ROWS
S_PART = S // WPR
K = 256
NCH = S_PART // K
DK = D // L
CHK = 512
NCHK = S // CHK
NEG = np.float32(-1e30)
TINY = np.float32(1e-20)
F32 = jnp.float32


def _sc_body(ld_hbm, conds_hbm, cmask_hbm, names_hbm, nmask_hbm, acts_hbm,
             outs_hbm, we_hbm, out_hbm, xacc_hbm, xms_hbm,
             buf0, buf1, xm_buf, mask_buf, ld_buf, we_buf, names_buf, outs_buf,
             nmask_buf, acc_buf, pacc_buf, ms_buf, pms_buf, o_buf, sem0, sem1):
    cidx = lax.axis_index("c")
    sidx = lax.axis_index("s")
    q = sidx % WPR
    b = cidx * (SC_ROWS // NC) + sidx // WPR
    w = cidx * NS + sidx
    s0 = q * S_PART
    lane = lax.iota(jnp.int32, L)

    pltpu.sync_copy(ld_hbm, ld_buf.at[pl.ds(0, B * OBS)])
    pltpu.sync_copy(we_hbm, we_buf)
    pltpu.sync_copy(cmask_hbm.at[b, pl.ds(s0, S_PART)], mask_buf)

    zeros_i = jnp.zeros((L,), jnp.int32)

    def obs_step(j, o):
        ldv = ld_buf[pl.ds(OBS * b + j, L)]
        sc = jnp.take(ldv, zeros_i)
        return tuple(o[k] + sc * we_buf[j, pl.ds(L * k, L)] for k in range(DK))

    obs = lax.fori_loop(0, OBS, obs_step,
                        tuple(jnp.zeros((L,), F32) for _ in range(DK)))

    def dma(src_hbm, ch, bufref, sem):
        return pltpu.make_async_copy(
            src_hbm.at[b, pl.ds(s0 + ch * K, K), :], bufref, sem)

    def compute_a(bufref, ch, mm):
        base = ch * K

        def group_a(g, mm_):
            tv = jnp.zeros((L,), F32)
            for j in range(L):
                i = g * L + j
                racc = bufref[i, pl.ds(0, L)] * obs[0]
                for k in range(1, DK):
                    racc = racc + bufref[i, pl.ds(L * k, L)] * obs[k]
                tv = jnp.where(lane == j, jnp.sum(racc), tv)
            mv = mask_buf[pl.ds(base + g * L, L)]
            xm = jnp.where(mv > 0, tv, NEG)
            xm_buf[pl.ds(base + g * L, L)] = xm
            return jnp.maximum(mm_, xm)

        return lax.fori_loop(0, K // L, group_a, mm)

    dma(conds_hbm, 0, buf0, sem0).start()
    dma(conds_hbm, 1, buf1, sem1).start()

    def outer_a(g2, mmax):
        for qq, (bufref, sem) in enumerate(((buf0, sem0), (buf1, sem1))):
            ch = 2 * g2 + qq
            dma(conds_hbm, ch, bufref, sem).wait()
            mmax = compute_a(bufref, ch, mmax)

            @pl.when(ch + 2 < NCH)
            def _():
                dma(conds_hbm, ch + 2, bufref, sem).start()
        return mmax

    mmax = lax.fori_loop(0, NCH // 2, outer_a, jnp.full((L,), NEG, F32))
    m_splat = jnp.full((L,), jnp.maximum(jnp.max(mmax), np.float32(0.0)), F32)

    def compute_b(bufref, ch, carry):
        base = ch * K

        def group_b(g, car):
            a = list(car[:DK])
            se = car[DK]
            xm = xm_buf[pl.ds(base + g * L, L)]
            mv = mask_buf[pl.ds(base + g * L, L)]
            e = jnp.exp(xm - m_splat) * mv
            se = se + e
            for j in range(L):
                i = g * L + j
                wj = e[j]
                for k in range(DK):
                    a[k] = a[k] + wj * bufref[i, pl.ds(L * k, L)]
            return (*a, se)

        return lax.fori_loop(0, K // L, group_b, carry)

    dma(acts_hbm, 0, buf0, sem0).start()
    dma(acts_hbm, 1, buf1, sem1).start()

    def outer_b(g2, carry):
        for qq, (bufref, sem) in enumerate(((buf0, sem0), (buf1, sem1))):
            ch = 2 * g2 + qq
            dma(acts_hbm, ch, bufref, sem).wait()
            carry = compute_b(bufref, ch, carry)

            @pl.when(ch + 2 < NCH)
            def _():
                dma(acts_hbm, ch + 2, bufref, sem).start()
        return carry

    init = tuple(jnp.zeros((L,), F32) for _ in range(DK + 1))
    res = lax.fori_loop(0, NCH // 2, outer_b, init)
    accs, sum_e = res[:DK], res[DK]

    for k in range(DK):
        acc_buf[pl.ds(L * k, L)] = accs[k]
    s_splat = jnp.full((L,), jnp.sum(sum_e), F32)
    ms_buf[pl.ds(0, L)] = m_splat
    ms_buf[pl.ds(L, L)] = s_splat
    pltpu.sync_copy(acc_buf, xacc_hbm.at[w])
    pltpu.sync_copy(ms_buf, xms_hbm.at[w])
    plsc.subcore_barrier()

    @pl.when(q == 0)
    def _stage_c():
        pltpu.sync_copy(names_hbm.at[b], names_buf)
        pltpu.sync_copy(nmask_hbm.at[b], nmask_buf)
        pltpu.sync_copy(outs_hbm.at[b], outs_buf)

        ms = [(m_splat, s_splat)]
        for p in range(1, WPR):
            pltpu.sync_copy(xms_hbm.at[w + p], pms_buf)
            ms.append((pms_buf[pl.ds(0, L)], pms_buf[pl.ds(L, L)]))
        mg = ms[0][0]
        for p in range(1, WPR):
            mg = jnp.maximum(mg, ms[p][0])
        rs = [jnp.exp(m_p - mg) for (m_p, _) in ms]
        s_tot = ms[0][1] * rs[0]
        for p in range(1, WPR):
            s_tot = s_tot + ms[p][1] * rs[p]
        denom = jnp.maximum(s_tot, TINY)

        act = [accs[k] * rs[0] for k in range(DK)]
        for p in range(1, WPR):
            pltpu.sync_copy(xacc_hbm.at[w + p], pacc_buf)
            for k in range(DK):
                act[k] = act[k] + pacc_buf[pl.ds(L * k, L)] * rs[p]
        act = [a_k / denom for a_k in act]

        def logit_step(c, carry):
            l0_, l1_ = carry
            lacc = names_buf[c, pl.ds(0, L)] * act[0]
            for k in range(1, DK):
                lacc = lacc + names_buf[c, pl.ds(L * k, L)] * act[k]
            t = jnp.sum(lacc)
            l0_ = jnp.where(lane == c, t, l0_)
            l1_ = jnp.where(lane == c - L, t, l1_)
            return (l0_, l1_)

        l0, l1 = lax.fori_loop(0, C, logit_step,
                               (jnp.zeros((L,), F32), jnp.zeros((L,), F32)))

        nm0 = nmask_buf[pl.ds(0, L)]
        nm1 = nmask_buf[pl.ds(L, L)]
        x0 = jnp.where(nm0 > 0, l0, NEG)
        x1 = jnp.where(nm1 > 0, l1, NEG)
        m = jnp.maximum(jnp.maximum(jnp.max(x0), jnp.max(x1)), np.float32(0.0))
        e0 = jnp.exp(x0 - m) * nm0
        e1 = jnp.exp(x1 - m) * nm1
        dn = jnp.maximum(jnp.sum(e0) + jnp.sum(e1), TINY)
        w0 = e0 / dn
        w1 = e1 / dn

        idx4 = lane // 4
        out16 = jnp.zeros((L,), F32)
        for k in range(DK):
            wsrc = w0 if k < DK // 2 else w1
            wo = (4 * k) % L
            wsel = jnp.where(idx4 == 0, wsrc[wo],
                   jnp.where(idx4 == 1, wsrc[wo + 1],
                   jnp.where(idx4 == 2, wsrc[wo + 2], wsrc[wo + 3])))
            out16 = out16 + wsel * outs_buf[pl.ds(L * k, L)]
        r = jnp.zeros((L,), F32)
        for a_i in range(A):
            v = out16[a_i] + out16[4 + a_i] + out16[8 + a_i] + out16[12 + a_i]
            r = jnp.where(lane == a_i, v, r)
        o_buf[...] = r
        pltpu.sync_copy(o_buf, out_hbm.at[b])


def _tc_body(ldT_ref, weT_ref, conds_ref, cmaskT_ref, names_ref, nmaskT_ref,
             acts_ref, outs_ref, o_ref, obs_scr, acc_scr, ms_scr):
    i = pl.program_id(0)
    j = pl.program_id(1)
    row = SC_ROWS + i

    @pl.when((i == 0) & (j == 0))
    def _once():
        obs_scr[...] = jnp.dot(weT_ref[...], ldT_ref[...],
                               preferred_element_type=F32)

    @pl.when(j == 0)
    def _init():
        acc_scr[...] = jnp.zeros((B, D), F32)
        ms_scr[0] = 0.0
        ms_scr[1] = 0.0

    onehot = (lax.broadcasted_iota(jnp.int32, (1, B), 1) == row).astype(F32)
    conds2d = conds_ref[0]
    acts2d = acts_ref[0]
    tv16 = jnp.dot(conds2d, obs_scr[...],
                   preferred_element_type=F32)
    mb = cmaskT_ref[...] * onehot
    xm = jnp.where(mb > 0, tv16, NEG)
    m_old = ms_scr[0]
    m_new = jnp.maximum(m_old, jnp.max(xm))
    rsc = jnp.exp(m_old - m_new)
    e16 = jnp.exp(xm - m_new) * mb
    ms_scr[0] = m_new
    ms_scr[1] = ms_scr[1] * rsc + jnp.sum(e16)
    acc_scr[...] = acc_scr[...] * rsc + lax.dot_general(
        e16, acts2d, (((0,), (0,)), ((), ())),
        preferred_element_type=F32)

    @pl.when(j == NCHK - 1)
    def _final():
        action = (jnp.sum(acc_scr[...], axis=0, keepdims=True)
                  / jnp.maximum(ms_scr[1], TINY))
        logits = jnp.sum(names_ref[0] * action, axis=1,
                         keepdims=True)
        nm = jnp.sum(nmaskT_ref[...] * onehot, axis=1, keepdims=True)
        xn = jnp.where(nm > 0, logits, NEG)
        mn = jnp.maximum(jnp.max(xn), np.float32(0.0))
        en = jnp.exp(xn - mn) * nm
        wn = en / jnp.maximum(jnp.sum(en), TINY)
        o_ref[pl.ds(i, 1), :] = jnp.sum(wn * outs_ref[0], axis=0,
                                        keepdims=True)


@jax.jit
def _cond_agent(ld_flat, ld_t, conds, cmask_f, cmask_t, names, nmask_f,
                nmask_t, acts, outs_flat, outs, w_eval, we_t):
    mesh = plsc.VectorSubcoreMesh(core_axis_name="c", subcore_axis_name="s",
                                  num_cores=NC, num_subcores=NS)
    sc_fn = pl.kernel(
        _sc_body,
        out_type=(jax.ShapeDtypeStruct((B, L), F32),
                  jax.ShapeDtypeStruct((NW, D), F32),
                  jax.ShapeDtypeStruct((NW, 2 * L), F32)),
        mesh=mesh,
        compiler_params=pltpu.CompilerParams(needs_layout_passes=False),
        scratch_types=[
            pltpu.VMEM((K, D), F32),
            pltpu.VMEM((K, D), F32),
            pltpu.VMEM((S_PART,), F32),
            pltpu.VMEM((S_PART,), F32),
            pltpu.VMEM((B * OBS + L,), F32),
            pltpu.VMEM((OBS, D), F32),
            pltpu.VMEM((C, D), F32),
            pltpu.VMEM((C * A,), F32),
            pltpu.VMEM((C,), F32),
            pltpu.VMEM((D,), F32),
            pltpu.VMEM((D,), F32),
            pltpu.VMEM((2 * L,), F32),
            pltpu.VMEM((2 * L,), F32),
            pltpu.VMEM((L,), F32),
            pltpu.SemaphoreType.DMA,
            pltpu.SemaphoreType.DMA,
        ],
    )
    out_sc, _, _ = sc_fn(ld_flat, conds, cmask_f, names, nmask_f, acts,
                         outs_flat, w_eval)

    out_tc = pl.pallas_call(
        _tc_body,
        grid=(TC_ROWS, NCHK),
        in_specs=[
            pl.BlockSpec((OBS, B), lambda i, j: (0, 0)),
            pl.BlockSpec((D, OBS), lambda i, j: (0, 0)),
            pl.BlockSpec((1, CHK, D), lambda i, j: (SC_ROWS + i, j, 0)),
            pl.BlockSpec((CHK, B), lambda i, j: (j, 0)),
            pl.BlockSpec((1, C, D), lambda i, j: (SC_ROWS + i, 0, 0)),
            pl.BlockSpec((C, B), lambda i, j: (0, 0)),
            pl.BlockSpec((1, CHK, D), lambda i, j: (SC_ROWS + i, j, 0)),
            pl.BlockSpec((1, C, A), lambda i, j: (SC_ROWS + i, 0, 0)),
        ],
        out_specs=pl.BlockSpec((TC_ROWS, A), lambda i, j: (0, 0)),
        out_shape=jax.ShapeDtypeStruct((TC_ROWS, A), F32),
        scratch_shapes=[
            pltpu.VMEM((D, B), F32),
            pltpu.VMEM((B, D), F32),
            pltpu.SMEM((2,), F32),
        ],
        compiler_params=pltpu.CompilerParams(
            dimension_semantics=("arbitrary", "arbitrary")),
    )(ld_t, we_t, conds, cmask_t, names, nmask_t, acts, outs)
    return out_sc, out_tc


def kernel(low_dim, conds_padded, conds_mask, names_padded, names_mask,
           actions_padded, outputs_padded, W_eval):
    ld_flat = low_dim.reshape(B * OBS)
    cmask_f = conds_mask.astype(jnp.float32)
    nmask_f = names_mask.astype(jnp.float32)
    cmask_t = cmask_f.T
    nmask_t = nmask_f.T
    outs_flat = outputs_padded.reshape(B, C * A)
    out_sc, out_tc = _cond_agent(ld_flat, low_dim.T, conds_padded, cmask_f,
                                 cmask_t, names_padded, nmask_f, nmask_t,
                                 actions_padded, outs_flat, outputs_padded,
                                 W_eval, W_eval.T)
    return jnp.concatenate([out_sc[:SC_ROWS, :A], out_tc], axis=0)


# speedup vs baseline: 1.5917x; 1.5917x over previous
import jax
import jax.numpy as jnp
import numpy as np
from jax import lax
from jax.experimental import pallas as pl
from jax.experimental.pallas import tpu as pltpu
from jax.experimental.pallas import tpu_sc as plsc

B, S, D = 16, 4096, 128
OBS, C, A = 39, 32, 4
NC, NS, L = 2, 16, 16
NW = NC * NS
SC_ROWS = 4
TC_ROWS = B - SC_ROWS
WPR = NW // SC_ROWS
S_PART = S // WPR
K = 256
NCH = S_PART // K
DK = D // L
CHK = 2048
NCHK = S // CHK
NEG = np.float32(-1e30)
TINY = np.float32(1e-20)
F32 = jnp.float32


def _sc_body(ld_hbm, conds_hbm, cmask_hbm, names_hbm, nmask_hbm, acts_hbm,
             outs_hbm, we_hbm, out_hbm, xacc_hbm, xms_hbm,
             buf0, buf1, xm_buf, mask_buf, ld_buf, we_buf, names_buf, outs_buf,
             nmask_buf, acc_buf, pacc_buf, ms_buf, pms_buf, o_buf, sem0, sem1):
    cidx = lax.axis_index("c")
    sidx = lax.axis_index("s")
    q = sidx % WPR
    b = cidx * (SC_ROWS // NC) + sidx // WPR
    w = cidx * NS + sidx
    s0 = q * S_PART
    lane = lax.iota(jnp.int32, L)

    pltpu.sync_copy(ld_hbm, ld_buf.at[pl.ds(0, B * OBS)])
    pltpu.sync_copy(we_hbm, we_buf)
    pltpu.sync_copy(cmask_hbm.at[b, pl.ds(s0, S_PART)], mask_buf)

    zeros_i = jnp.zeros((L,), jnp.int32)

    def obs_step(j, o):
        ldv = ld_buf[pl.ds(OBS * b + j, L)]
        sc = jnp.take(ldv, zeros_i)
        return tuple(o[k] + sc * we_buf[j, pl.ds(L * k, L)] for k in range(DK))

    obs = lax.fori_loop(0, OBS, obs_step,
                        tuple(jnp.zeros((L,), F32) for _ in range(DK)))

    def dma(src_hbm, ch, bufref, sem):
        return pltpu.make_async_copy(
            src_hbm.at[b, pl.ds(s0 + ch * K, K), :], bufref, sem)

    def compute_a(bufref, ch, mm):
        base = ch * K

        def group_a(g, mm_):
            tv = jnp.zeros((L,), F32)
            for j in range(L):
                i = g * L + j
                racc = bufref[i, pl.ds(0, L)] * obs[0]
                for k in range(1, DK):
                    racc = racc + bufref[i, pl.ds(L * k, L)] * obs[k]
                tv = jnp.where(lane == j, jnp.sum(racc), tv)
            mv = mask_buf[pl.ds(base + g * L, L)]
            xm = jnp.where(mv > 0, tv, NEG)
            xm_buf[pl.ds(base + g * L, L)] = xm
            return jnp.maximum(mm_, xm)

        return lax.fori_loop(0, K // L, group_a, mm)

    dma(conds_hbm, 0, buf0, sem0).start()
    dma(conds_hbm, 1, buf1, sem1).start()

    def outer_a(g2, mmax):
        for qq, (bufref, sem) in enumerate(((buf0, sem0), (buf1, sem1))):
            ch = 2 * g2 + qq
            dma(conds_hbm, ch, bufref, sem).wait()
            mmax = compute_a(bufref, ch, mmax)

            @pl.when(ch + 2 < NCH)
            def _():
                dma(conds_hbm, ch + 2, bufref, sem).start()
        return mmax

    mmax = lax.fori_loop(0, NCH // 2, outer_a, jnp.full((L,), NEG, F32))
    m_splat = jnp.full((L,), jnp.maximum(jnp.max(mmax), np.float32(0.0)), F32)

    def compute_b(bufref, ch, carry):
        base = ch * K

        def group_b(g, car):
            a = list(car[:DK])
            se = car[DK]
            xm = xm_buf[pl.ds(base + g * L, L)]
            mv = mask_buf[pl.ds(base + g * L, L)]
            e = jnp.exp(xm - m_splat) * mv
            se = se + e
            for j in range(L):
                i = g * L + j
                wj = e[j]
                for k in range(DK):
                    a[k] = a[k] + wj * bufref[i, pl.ds(L * k, L)]
            return (*a, se)

        return lax.fori_loop(0, K // L, group_b, carry)

    dma(acts_hbm, 0, buf0, sem0).start()
    dma(acts_hbm, 1, buf1, sem1).start()

    def outer_b(g2, carry):
        for qq, (bufref, sem) in enumerate(((buf0, sem0), (buf1, sem1))):
            ch = 2 * g2 + qq
            dma(acts_hbm, ch, bufref, sem).wait()
            carry = compute_b(bufref, ch, carry)

            @pl.when(ch + 2 < NCH)
            def _():
                dma(acts_hbm, ch + 2, bufref, sem).start()
        return carry

    init = tuple(jnp.zeros((L,), F32) for _ in range(DK + 1))
    res = lax.fori_loop(0, NCH // 2, outer_b, init)
    accs, sum_e = res[:DK], res[DK]

    for k in range(DK):
        acc_buf[pl.ds(L * k, L)] = accs[k]
    s_splat = jnp.full((L,), jnp.sum(sum_e), F32)
    ms_buf[pl.ds(0, L)] = m_splat
    ms_buf[pl.ds(L, L)] = s_splat
    pltpu.sync_copy(acc_buf, xacc_hbm.at[w])
    pltpu.sync_copy(ms_buf, xms_hbm.at[w])
    plsc.subcore_barrier()

    @pl.when(q == 0)
    def _stage_c():
        pltpu.sync_copy(names_hbm.at[b], names_buf)
        pltpu.sync_copy(nmask_hbm.at[b], nmask_buf)
        pltpu.sync_copy(outs_hbm.at[b], outs_buf)

        ms = [(m_splat, s_splat)]
        for p in range(1, WPR):
            pltpu.sync_copy(xms_hbm.at[w + p], pms_buf)
            ms.append((pms_buf[pl.ds(0, L)], pms_buf[pl.ds(L, L)]))
        mg = ms[0][0]
        for p in range(1, WPR):
            mg = jnp.maximum(mg, ms[p][0])
        rs = [jnp.exp(m_p - mg) for (m_p, _) in ms]
        s_tot = ms[0][1] * rs[0]
        for p in range(1, WPR):
            s_tot = s_tot + ms[p][1] * rs[p]
        denom = jnp.maximum(s_tot, TINY)

        act = [accs[k] * rs[0] for k in range(DK)]
        for p in range(1, WPR):
            pltpu.sync_copy(xacc_hbm.at[w + p], pacc_buf)
            for k in range(DK):
                act[k] = act[k] + pacc_buf[pl.ds(L * k, L)] * rs[p]
        act = [a_k / denom for a_k in act]

        def logit_step(c, carry):
            l0_, l1_ = carry
            lacc = names_buf[c, pl.ds(0, L)] * act[0]
            for k in range(1, DK):
                lacc = lacc + names_buf[c, pl.ds(L * k, L)] * act[k]
            t = jnp.sum(lacc)
            l0_ = jnp.where(lane == c, t, l0_)
            l1_ = jnp.where(lane == c - L, t, l1_)
            return (l0_, l1_)

        l0, l1 = lax.fori_loop(0, C, logit_step,
                               (jnp.zeros((L,), F32), jnp.zeros((L,), F32)))

        nm0 = nmask_buf[pl.ds(0, L)]
        nm1 = nmask_buf[pl.ds(L, L)]
        x0 = jnp.where(nm0 > 0, l0, NEG)
        x1 = jnp.where(nm1 > 0, l1, NEG)
        m = jnp.maximum(jnp.maximum(jnp.max(x0), jnp.max(x1)), np.float32(0.0))
        e0 = jnp.exp(x0 - m) * nm0
        e1 = jnp.exp(x1 - m) * nm1
        dn = jnp.maximum(jnp.sum(e0) + jnp.sum(e1), TINY)
        w0 = e0 / dn
        w1 = e1 / dn

        idx4 = lane // 4
        out16 = jnp.zeros((L,), F32)
        for k in range(DK):
            wsrc = w0 if k < DK // 2 else w1
            wo = (4 * k) % L
            wsel = jnp.where(idx4 == 0, wsrc[wo],
                   jnp.where(idx4 == 1, wsrc[wo + 1],
                   jnp.where(idx4 == 2, wsrc[wo + 2], wsrc[wo + 3])))
            out16 = out16 + wsel * outs_buf[pl.ds(L * k, L)]
        r = jnp.zeros((L,), F32)
        for a_i in range(A):
            v = out16[a_i] + out16[4 + a_i] + out16[8 + a_i] + out16[12 + a_i]
            r = jnp.where(lane == a_i, v, r)
        o_buf[...] = r
        pltpu.sync_copy(o_buf, out_hbm.at[b])


def _tc_body(ldT_ref, weT_ref, conds_ref, cmaskT_ref, names_ref, nmaskT_ref,
             acts_ref, outs_ref, o_ref, obs_scr, acc_scr, ms_scr):
    i = pl.program_id(0)
    j = pl.program_id(1)
    row = SC_ROWS + i

    @pl.when((i == 0) & (j == 0))
    def _once():
        obs_scr[...] = jnp.dot(weT_ref[...], ldT_ref[...],
                               preferred_element_type=F32)

    @pl.when(j == 0)
    def _init():
        acc_scr[...] = jnp.zeros((B, D), F32)
        ms_scr[0] = 0.0
        ms_scr[1] = 0.0

    onehot = (lax.broadcasted_iota(jnp.int32, (1, B), 1) == row).astype(F32)
    conds2d = conds_ref[0]
    acts2d = acts_ref[0]
    tv16 = jnp.dot(conds2d, obs_scr[...],
                   preferred_element_type=F32)
    mb = cmaskT_ref[...] * onehot
    xm = jnp.where(mb > 0, tv16, NEG)
    m_old = ms_scr[0]
    m_new = jnp.maximum(m_old, jnp.max(xm))
    rsc = jnp.exp(m_old - m_new)
    e16 = jnp.exp(xm - m_new) * mb
    ms_scr[0] = m_new
    ms_scr[1] = ms_scr[1] * rsc + jnp.sum(e16)
    acc_scr[...] = acc_scr[...] * rsc + lax.dot_general(
        e16, acts2d, (((0,), (0,)), ((), ())),
        preferred_element_type=F32)

    @pl.when(j == NCHK - 1)
    def _final():
        action = (jnp.sum(acc_scr[...], axis=0, keepdims=True)
                  / jnp.maximum(ms_scr[1], TINY))
        logits = jnp.sum(names_ref[0] * action, axis=1,
                         keepdims=True)
        nm = jnp.sum(nmaskT_ref[...] * onehot, axis=1, keepdims=True)
        xn = jnp.where(nm > 0, logits, NEG)
        mn = jnp.maximum(jnp.max(xn), np.float32(0.0))
        en = jnp.exp(xn - mn) * nm
        wn = en / jnp.maximum(jnp.sum(en), TINY)
        o_ref[pl.ds(i, 1), :] = jnp.sum(wn * outs_ref[0], axis=0,
                                        keepdims=True)


@jax.jit
def _cond_agent(ld_flat, ld_t, conds, cmask_f, cmask_t, names, nmask_f,
                nmask_t, acts, outs_flat, outs, w_eval, we_t):
    mesh = plsc.VectorSubcoreMesh(core_axis_name="c", subcore_axis_name="s",
                                  num_cores=NC, num_subcores=NS)
    sc_fn = pl.kernel(
        _sc_body,
        out_type=(jax.ShapeDtypeStruct((B, L), F32),
                  jax.ShapeDtypeStruct((NW, D), F32),
                  jax.ShapeDtypeStruct((NW, 2 * L), F32)),
        mesh=mesh,
        compiler_params=pltpu.CompilerParams(needs_layout_passes=False),
        scratch_types=[
            pltpu.VMEM((K, D), F32),
            pltpu.VMEM((K, D), F32),
            pltpu.VMEM((S_PART,), F32),
            pltpu.VMEM((S_PART,), F32),
            pltpu.VMEM((B * OBS + L,), F32),
            pltpu.VMEM((OBS, D), F32),
            pltpu.VMEM((C, D), F32),
            pltpu.VMEM((C * A,), F32),
            pltpu.VMEM((C,), F32),
            pltpu.VMEM((D,), F32),
            pltpu.VMEM((D,), F32),
            pltpu.VMEM((2 * L,), F32),
            pltpu.VMEM((2 * L,), F32),
            pltpu.VMEM((L,), F32),
            pltpu.SemaphoreType.DMA,
            pltpu.SemaphoreType.DMA,
        ],
    )
    out_sc, _, _ = sc_fn(ld_flat, conds, cmask_f, names, nmask_f, acts,
                         outs_flat, w_eval)

    out_tc = pl.pallas_call(
        _tc_body,
        grid=(TC_ROWS, NCHK),
        in_specs=[
            pl.BlockSpec((OBS, B), lambda i, j: (0, 0)),
            pl.BlockSpec((D, OBS), lambda i, j: (0, 0)),
            pl.BlockSpec((1, CHK, D), lambda i, j: (SC_ROWS + i, j, 0)),
            pl.BlockSpec((CHK, B), lambda i, j: (j, 0)),
            pl.BlockSpec((1, C, D), lambda i, j: (SC_ROWS + i, 0, 0)),
            pl.BlockSpec((C, B), lambda i, j: (0, 0)),
            pl.BlockSpec((1, CHK, D), lambda i, j: (SC_ROWS + i, j, 0)),
            pl.BlockSpec((1, C, A), lambda i, j: (SC_ROWS + i, 0, 0)),
        ],
        out_specs=pl.BlockSpec((TC_ROWS, A), lambda i, j: (0, 0)),
        out_shape=jax.ShapeDtypeStruct((TC_ROWS, A), F32),
        scratch_shapes=[
            pltpu.VMEM((D, B), F32),
            pltpu.VMEM((B, D), F32),
            pltpu.SMEM((2,), F32),
        ],
        compiler_params=pltpu.CompilerParams(
            dimension_semantics=("arbitrary", "arbitrary")),
    )(ld_t, we_t, conds, cmask_t, names, nmask_t, acts, outs)
    return out_sc, out_tc


def kernel(low_dim, conds_padded, conds_mask, names_padded, names_mask,
           actions_padded, outputs_padded, W_eval):
    ld_flat = low_dim.reshape(B * OBS)
    cmask_f = conds_mask.astype(jnp.float32)
    nmask_f = names_mask.astype(jnp.float32)
    cmask_t = cmask_f.T
    nmask_t = nmask_f.T
    outs_flat = outputs_padded.reshape(B, C * A)
    out_sc, out_tc = _cond_agent(ld_flat, low_dim.T, conds_padded, cmask_f,
                                 cmask_t, names_padded, nmask_f, nmask_t,
                                 actions_padded, outs_flat, outputs_padded,
                                 W_eval, W_eval.T)
    return jnp.concatenate([out_sc[:SC_ROWS, :A], out_tc], axis=0)


# speedup vs baseline: 1.8958x; 1.1910x over previous
import jax
import jax.numpy as jnp
import numpy as np
from jax import lax
from jax.experimental import pallas as pl
from jax.experimental.pallas import tpu as pltpu
from jax.experimental.pallas import tpu_sc as plsc

B, S, D = 16, 4096, 128
OBS, C, A = 39, 32, 4
NC, NS, L = 2, 16, 16
NW = NC * NS
SC_ROWS = 4
TC_ROWS = B - SC_ROWS
WPR = NW // SC_ROWS
S_PART = S // WPR
K = 256
NCH = S_PART // K
DK = D // L
CHK = 4096
NCHK = S // CHK
NEG = np.float32(-1e30)
TINY = np.float32(1e-20)
F32 = jnp.float32


def _sc_body(ld_hbm, conds_hbm, cmask_hbm, names_hbm, nmask_hbm, acts_hbm,
             outs_hbm, we_hbm, out_hbm, xacc_hbm, xms_hbm,
             buf0, buf1, xm_buf, mask_buf, ld_buf, we_buf, names_buf, outs_buf,
             nmask_buf, acc_buf, pacc_buf, ms_buf, pms_buf, o_buf, sem0, sem1):
    cidx = lax.axis_index("c")
    sidx = lax.axis_index("s")
    q = sidx % WPR
    b = cidx * (SC_ROWS // NC) + sidx // WPR
    w = cidx * NS + sidx
    s0 = q * S_PART
    lane = lax.iota(jnp.int32, L)

    pltpu.sync_copy(ld_hbm, ld_buf.at[pl.ds(0, B * OBS)])
    pltpu.sync_copy(we_hbm, we_buf)
    pltpu.sync_copy(cmask_hbm.at[b, pl.ds(s0, S_PART)], mask_buf)

    zeros_i = jnp.zeros((L,), jnp.int32)

    def obs_step(j, o):
        ldv = ld_buf[pl.ds(OBS * b + j, L)]
        sc = jnp.take(ldv, zeros_i)
        return tuple(o[k] + sc * we_buf[j, pl.ds(L * k, L)] for k in range(DK))

    obs = lax.fori_loop(0, OBS, obs_step,
                        tuple(jnp.zeros((L,), F32) for _ in range(DK)))

    def dma(src_hbm, ch, bufref, sem):
        return pltpu.make_async_copy(
            src_hbm.at[b, pl.ds(s0 + ch * K, K), :], bufref, sem)

    def compute_a(bufref, ch, mm):
        base = ch * K

        def group_a(g, mm_):
            tv = jnp.zeros((L,), F32)
            for j in range(L):
                i = g * L + j
                racc = bufref[i, pl.ds(0, L)] * obs[0]
                for k in range(1, DK):
                    racc = racc + bufref[i, pl.ds(L * k, L)] * obs[k]
                tv = jnp.where(lane == j, jnp.sum(racc), tv)
            mv = mask_buf[pl.ds(base + g * L, L)]
            xm = jnp.where(mv > 0, tv, NEG)
            xm_buf[pl.ds(base + g * L, L)] = xm
            return jnp.maximum(mm_, xm)

        return lax.fori_loop(0, K // L, group_a, mm)

    dma(conds_hbm, 0, buf0, sem0).start()
    dma(conds_hbm, 1, buf1, sem1).start()

    def outer_a(g2, mmax):
        for qq, (bufref, sem) in enumerate(((buf0, sem0), (buf1, sem1))):
            ch = 2 * g2 + qq
            dma(conds_hbm, ch, bufref, sem).wait()
            mmax = compute_a(bufref, ch, mmax)

            @pl.when(ch + 2 < NCH)
            def _():
                dma(conds_hbm, ch + 2, bufref, sem).start()
        return mmax

    mmax = lax.fori_loop(0, NCH // 2, outer_a, jnp.full((L,), NEG, F32))
    m_splat = jnp.full((L,), jnp.maximum(jnp.max(mmax), np.float32(0.0)), F32)

    def compute_b(bufref, ch, carry):
        base = ch * K

        def group_b(g, car):
            a = list(car[:DK])
            se = car[DK]
            xm = xm_buf[pl.ds(base + g * L, L)]
            mv = mask_buf[pl.ds(base + g * L, L)]
            e = jnp.exp(xm - m_splat) * mv
            se = se + e
            for j in range(L):
                i = g * L + j
                wj = e[j]
                for k in range(DK):
                    a[k] = a[k] + wj * bufref[i, pl.ds(L * k, L)]
            return (*a, se)

        return lax.fori_loop(0, K // L, group_b, carry)

    dma(acts_hbm, 0, buf0, sem0).start()
    dma(acts_hbm, 1, buf1, sem1).start()

    def outer_b(g2, carry):
        for qq, (bufref, sem) in enumerate(((buf0, sem0), (buf1, sem1))):
            ch = 2 * g2 + qq
            dma(acts_hbm, ch, bufref, sem).wait()
            carry = compute_b(bufref, ch, carry)

            @pl.when(ch + 2 < NCH)
            def _():
                dma(acts_hbm, ch + 2, bufref, sem).start()
        return carry

    init = tuple(jnp.zeros((L,), F32) for _ in range(DK + 1))
    res = lax.fori_loop(0, NCH // 2, outer_b, init)
    accs, sum_e = res[:DK], res[DK]

    for k in range(DK):
        acc_buf[pl.ds(L * k, L)] = accs[k]
    s_splat = jnp.full((L,), jnp.sum(sum_e), F32)
    ms_buf[pl.ds(0, L)] = m_splat
    ms_buf[pl.ds(L, L)] = s_splat
    pltpu.sync_copy(acc_buf, xacc_hbm.at[w])
    pltpu.sync_copy(ms_buf, xms_hbm.at[w])
    plsc.subcore_barrier()

    @pl.when(q == 0)
    def _stage_c():
        pltpu.sync_copy(names_hbm.at[b], names_buf)
        pltpu.sync_copy(nmask_hbm.at[b], nmask_buf)
        pltpu.sync_copy(outs_hbm.at[b], outs_buf)

        ms = [(m_splat, s_splat)]
        for p in range(1, WPR):
            pltpu.sync_copy(xms_hbm.at[w + p], pms_buf)
            ms.append((pms_buf[pl.ds(0, L)], pms_buf[pl.ds(L, L)]))
        mg = ms[0][0]
        for p in range(1, WPR):
            mg = jnp.maximum(mg, ms[p][0])
        rs = [jnp.exp(m_p - mg) for (m_p, _) in ms]
        s_tot = ms[0][1] * rs[0]
        for p in range(1, WPR):
            s_tot = s_tot + ms[p][1] * rs[p]
        denom = jnp.maximum(s_tot, TINY)

        act = [accs[k] * rs[0] for k in range(DK)]
        for p in range(1, WPR):
            pltpu.sync_copy(xacc_hbm.at[w + p], pacc_buf)
            for k in range(DK):
                act[k] = act[k] + pacc_buf[pl.ds(L * k, L)] * rs[p]
        act = [a_k / denom for a_k in act]

        def logit_step(c, carry):
            l0_, l1_ = carry
            lacc = names_buf[c, pl.ds(0, L)] * act[0]
            for k in range(1, DK):
                lacc = lacc + names_buf[c, pl.ds(L * k, L)] * act[k]
            t = jnp.sum(lacc)
            l0_ = jnp.where(lane == c, t, l0_)
            l1_ = jnp.where(lane == c - L, t, l1_)
            return (l0_, l1_)

        l0, l1 = lax.fori_loop(0, C, logit_step,
                               (jnp.zeros((L,), F32), jnp.zeros((L,), F32)))

        nm0 = nmask_buf[pl.ds(0, L)]
        nm1 = nmask_buf[pl.ds(L, L)]
        x0 = jnp.where(nm0 > 0, l0, NEG)
        x1 = jnp.where(nm1 > 0, l1, NEG)
        m = jnp.maximum(jnp.maximum(jnp.max(x0), jnp.max(x1)), np.float32(0.0))
        e0 = jnp.exp(x0 - m) * nm0
        e1 = jnp.exp(x1 - m) * nm1
        dn = jnp.maximum(jnp.sum(e0) + jnp.sum(e1), TINY)
        w0 = e0 / dn
        w1 = e1 / dn

        idx4 = lane // 4
        out16 = jnp.zeros((L,), F32)
        for k in range(DK):
            wsrc = w0 if k < DK // 2 else w1
            wo = (4 * k) % L
            wsel = jnp.where(idx4 == 0, wsrc[wo],
                   jnp.where(idx4 == 1, wsrc[wo + 1],
                   jnp.where(idx4 == 2, wsrc[wo + 2], wsrc[wo + 3])))
            out16 = out16 + wsel * outs_buf[pl.ds(L * k, L)]
        r = jnp.zeros((L,), F32)
        for a_i in range(A):
            v = out16[a_i] + out16[4 + a_i] + out16[8 + a_i] + out16[12 + a_i]
            r = jnp.where(lane == a_i, v, r)
        o_buf[...] = r
        pltpu.sync_copy(o_buf, out_hbm.at[b])


def _tc_body(ldT_ref, weT_ref, conds_ref, cmaskT_ref, names_ref, nmaskT_ref,
             acts_ref, outs_ref, o_ref, obs_scr, acc_scr, ms_scr):
    i = pl.program_id(0)
    j = pl.program_id(1)
    row = SC_ROWS + i

    @pl.when((i == 0) & (j == 0))
    def _once():
        obs_scr[...] = jnp.dot(weT_ref[...], ldT_ref[...],
                               preferred_element_type=F32)

    @pl.when(j == 0)
    def _init():
        acc_scr[...] = jnp.zeros((B, D), F32)
        ms_scr[0] = 0.0
        ms_scr[1] = 0.0

    onehot = (lax.broadcasted_iota(jnp.int32, (1, B), 1) == row).astype(F32)
    conds2d = conds_ref[0]
    acts2d = acts_ref[0]
    tv16 = jnp.dot(conds2d, obs_scr[...],
                   preferred_element_type=F32)
    mb = cmaskT_ref[...] * onehot
    xm = jnp.where(mb > 0, tv16, NEG)
    m_old = ms_scr[0]
    m_new = jnp.maximum(m_old, jnp.max(xm))
    rsc = jnp.exp(m_old - m_new)
    e16 = jnp.exp(xm - m_new) * mb
    ms_scr[0] = m_new
    ms_scr[1] = ms_scr[1] * rsc + jnp.sum(e16)
    acc_scr[...] = acc_scr[...] * rsc + lax.dot_general(
        e16, acts2d, (((0,), (0,)), ((), ())),
        preferred_element_type=F32)

    @pl.when(j == NCHK - 1)
    def _final():
        action = (jnp.sum(acc_scr[...], axis=0, keepdims=True)
                  / jnp.maximum(ms_scr[1], TINY))
        logits = jnp.sum(names_ref[0] * action, axis=1,
                         keepdims=True)
        nm = jnp.sum(nmaskT_ref[...] * onehot, axis=1, keepdims=True)
        xn = jnp.where(nm > 0, logits, NEG)
        mn = jnp.maximum(jnp.max(xn), np.float32(0.0))
        en = jnp.exp(xn - mn) * nm
        wn = en / jnp.maximum(jnp.sum(en), TINY)
        o_ref[pl.ds(i, 1), :] = jnp.sum(wn * outs_ref[0], axis=0,
                                        keepdims=True)


@jax.jit
def _cond_agent(ld_flat, ld_t, conds, cmask_f, cmask_t, names, nmask_f,
                nmask_t, acts, outs_flat, outs, w_eval, we_t):
    mesh = plsc.VectorSubcoreMesh(core_axis_name="c", subcore_axis_name="s",
                                  num_cores=NC, num_subcores=NS)
    sc_fn = pl.kernel(
        _sc_body,
        out_type=(jax.ShapeDtypeStruct((B, L), F32),
                  jax.ShapeDtypeStruct((NW, D), F32),
                  jax.ShapeDtypeStruct((NW, 2 * L), F32)),
        mesh=mesh,
        compiler_params=pltpu.CompilerParams(needs_layout_passes=False),
        scratch_types=[
            pltpu.VMEM((K, D), F32),
            pltpu.VMEM((K, D), F32),
            pltpu.VMEM((S_PART,), F32),
            pltpu.VMEM((S_PART,), F32),
            pltpu.VMEM((B * OBS + L,), F32),
            pltpu.VMEM((OBS, D), F32),
            pltpu.VMEM((C, D), F32),
            pltpu.VMEM((C * A,), F32),
            pltpu.VMEM((C,), F32),
            pltpu.VMEM((D,), F32),
            pltpu.VMEM((D,), F32),
            pltpu.VMEM((2 * L,), F32),
            pltpu.VMEM((2 * L,), F32),
            pltpu.VMEM((L,), F32),
            pltpu.SemaphoreType.DMA,
            pltpu.SemaphoreType.DMA,
        ],
    )
    out_sc, _, _ = sc_fn(ld_flat, conds, cmask_f, names, nmask_f, acts,
                         outs_flat, w_eval)

    out_tc = pl.pallas_call(
        _tc_body,
        grid=(TC_ROWS, NCHK),
        in_specs=[
            pl.BlockSpec((OBS, B), lambda i, j: (0, 0)),
            pl.BlockSpec((D, OBS), lambda i, j: (0, 0)),
            pl.BlockSpec((1, CHK, D), lambda i, j: (SC_ROWS + i, j, 0)),
            pl.BlockSpec((CHK, B), lambda i, j: (j, 0)),
            pl.BlockSpec((1, C, D), lambda i, j: (SC_ROWS + i, 0, 0)),
            pl.BlockSpec((C, B), lambda i, j: (0, 0)),
            pl.BlockSpec((1, CHK, D), lambda i, j: (SC_ROWS + i, j, 0)),
            pl.BlockSpec((1, C, A), lambda i, j: (SC_ROWS + i, 0, 0)),
        ],
        out_specs=pl.BlockSpec((TC_ROWS, A), lambda i, j: (0, 0)),
        out_shape=jax.ShapeDtypeStruct((TC_ROWS, A), F32),
        scratch_shapes=[
            pltpu.VMEM((D, B), F32),
            pltpu.VMEM((B, D), F32),
            pltpu.SMEM((2,), F32),
        ],
        compiler_params=pltpu.CompilerParams(
            dimension_semantics=("arbitrary", "arbitrary"),
            vmem_limit_bytes=100 * 1024 * 1024),
    )(ld_t, we_t, conds, cmask_t, names, nmask_t, acts, outs)
    return out_sc, out_tc


def kernel(low_dim, conds_padded, conds_mask, names_padded, names_mask,
           actions_padded, outputs_padded, W_eval):
    ld_flat = low_dim.reshape(B * OBS)
    cmask_f = conds_mask.astype(jnp.float32)
    nmask_f = names_mask.astype(jnp.float32)
    cmask_t = cmask_f.T
    nmask_t = nmask_f.T
    outs_flat = outputs_padded.reshape(B, C * A)
    out_sc, out_tc = _cond_agent(ld_flat, low_dim.T, conds_padded, cmask_f,
                                 cmask_t, names_padded, nmask_f, nmask_t,
                                 actions_padded, outs_flat, outputs_padded,
                                 W_eval, W_eval.T)
    return jnp.concatenate([out_sc[:SC_ROWS, :A], out_tc], axis=0)
